# baseline (speedup 1.0000x reference)
import functools

import jax
import jax.numpy as jnp
from jax import lax
from jax.experimental import pallas as pl
from jax.experimental.pallas import tpu as pltpu
from jax.experimental.pallas import tpu_sc as plsc

_CHUNK = 128
_L = 16


def _make_gather(B, D, num_cores, num_subcores):
    nw = num_cores * num_subcores
    b_per_w = B // nw
    n_chunks = b_per_w // _CHUNK
    mesh = plsc.VectorSubcoreMesh(core_axis_name="c", subcore_axis_name="s")

    @functools.partial(
        pl.kernel,
        mesh=mesh,
        compiler_params=pltpu.CompilerParams(needs_layout_passes=False),
        out_type=jax.ShapeDtypeStruct((B, D), jnp.float32),
        scratch_types=[
            pltpu.VMEM((b_per_w,), jnp.int32),
            pltpu.VMEM((b_per_w,), jnp.int32),
            pltpu.VMEM((_CHUNK, 2 * D), jnp.float32),
            pltpu.VMEM((b_per_w, D), jnp.float32),
            pltpu.SemaphoreType.DMA,
        ],
    )
    def gather(idx_hbm, table_hbm, out_hbm, idx_v, pair_idx_v, pairs_v, sel_v,
               sem):
        wid = lax.axis_index("s") * num_cores + lax.axis_index("c")
        base = wid * b_per_w
        pltpu.async_copy(idx_hbm.at[pl.ds(base, b_per_w)], idx_v, sem).wait()

        def to_pair(g, _):
            v = idx_v[pl.ds(g * _L, _L)]
            pair_idx_v[pl.ds(g * _L, _L)] = v >> 1
            return 0

        lax.fori_loop(0, b_per_w // _L, to_pair, 0)

        iota = lax.iota(jnp.int32, _L)

        for j in range(n_chunks):
            pltpu.async_copy(
                table_hbm.at[pair_idx_v.at[pl.ds(j * _CHUNK, _CHUNK)]],
                pairs_v, sem,
            ).wait()

            def select(g, _, j=j):
                off = j * _CHUNK + g * _L
                loc_vec = g * _L + iota
                row_vec = off + iota
                half_vec = (idx_v[pl.ds(off, _L)] & 1) * D
                for c in range(D):
                    cv = jnp.full((_L,), c, jnp.int32)
                    v = plsc.load_gather(pairs_v, [loc_vec, half_vec + cv])
                    plsc.store_scatter(sel_v, [row_vec, cv], v)
                return 0

            lax.fori_loop(0, _CHUNK // _L, select, 0)

        pltpu.async_copy(sel_v, out_hbm.at[pl.ds(base, b_per_w)], sem).wait()

    return gather


def _proj_body(x_ref, wt_ref, b_ref, out_ref):
    out_ref[...] = (
        jnp.dot(x_ref[...], wt_ref[...], preferred_element_type=jnp.float32)
        + b_ref[...]
    )


def _proj(embs, Wt, b2d):
    B, D = embs.shape
    O = Wt.shape[1]
    blk = 2048
    return pl.pallas_call(
        _proj_body,
        grid=(B // blk,),
        in_specs=[
            pl.BlockSpec((blk, D), lambda i: (i, 0)),
            pl.BlockSpec((D, O), lambda i: (0, 0)),
            pl.BlockSpec((1, O), lambda i: (0, 0)),
        ],
        out_specs=pl.BlockSpec((blk, O), lambda i: (i, 0)),
        out_shape=jax.ShapeDtypeStruct((B, O), jnp.float32),
    )(embs, Wt, b2d)


def kernel(indices, table, W, b):
    info = plsc.get_sparse_core_info()
    V, D = table.shape
    table2 = table.reshape(V // 2, 2 * D)
    embs = _make_gather(indices.shape[0], D,
                        info.num_cores, info.num_subcores)(
        indices.astype(jnp.int32), table2
    )
    return _proj(embs, W.T, b.reshape(1, -1))


# speedup vs baseline: 1.0500x; 1.0500x over previous
import functools

import jax
import jax.numpy as jnp
from jax import lax
from jax.experimental import pallas as pl
from jax.experimental.pallas import tpu as pltpu
from jax.experimental.pallas import tpu_sc as plsc

_G = 16


def _make_gather(B, D, num_cores, num_subcores):
    nw = num_cores * num_subcores
    b_per_w = B // nw
    n_groups = b_per_w // _G
    mesh = plsc.VectorSubcoreMesh(core_axis_name="c", subcore_axis_name="s")

    @functools.partial(
        pl.kernel,
        mesh=mesh,
        compiler_params=pltpu.CompilerParams(needs_layout_passes=False),
        out_type=jax.ShapeDtypeStruct((B, D), jnp.float32),
        scratch_types=[
            pltpu.VMEM((b_per_w,), jnp.int32),
            pltpu.VMEM((b_per_w, D), jnp.float32),
            pltpu.SemaphoreType.DMA,
            pltpu.SemaphoreType.DMA,
        ],
    )
    def gather(idx_hbm, flat_hbm, out_hbm, idx_v, rows_v, isem, sem):
        wid = lax.axis_index("s") * num_cores + lax.axis_index("c")
        base = wid * b_per_w
        pltpu.async_copy(idx_hbm.at[pl.ds(base, b_per_w)], idx_v, isem).wait()

        def fire(g):
            v = idx_v[pl.ds(g * _G, _G)]
            return [
                pltpu.async_copy(
                    flat_hbm.at[pl.ds(v[i] * D, D)],
                    rows_v.at[g * _G + i],
                    sem,
                )
                for i in range(_G)
            ]

        pending = fire(0)
        for g in range(n_groups):
            nxt = fire(g + 1) if g + 1 < n_groups else []
            for c in pending:
                c.wait()
            pending = nxt

        pltpu.async_copy(rows_v, out_hbm.at[pl.ds(base, b_per_w)], isem).wait()

    return gather


def _proj_body(x_ref, wt_ref, b_ref, out_ref):
    out_ref[...] = (
        jnp.dot(x_ref[...], wt_ref[...], preferred_element_type=jnp.float32)
        + b_ref[...]
    )


def _proj(embs, Wt, b2d):
    B, D = embs.shape
    O = Wt.shape[1]
    blk = 2048
    return pl.pallas_call(
        _proj_body,
        grid=(B // blk,),
        in_specs=[
            pl.BlockSpec((blk, D), lambda i: (i, 0)),
            pl.BlockSpec((D, O), lambda i: (0, 0)),
            pl.BlockSpec((1, O), lambda i: (0, 0)),
        ],
        out_specs=pl.BlockSpec((blk, O), lambda i: (i, 0)),
        out_shape=jax.ShapeDtypeStruct((B, O), jnp.float32),
    )(embs, Wt, b2d)


def kernel(indices, table, W, b):
    info = plsc.get_sparse_core_info()
    V, D = table.shape
    flat = table.reshape(V * D)
    embs = _make_gather(indices.shape[0], D,
                        info.num_cores, info.num_subcores)(
        indices.astype(jnp.int32), flat
    )
    return _proj(embs, W.T, b.reshape(1, -1))


# speedup vs baseline: 1.7462x; 1.6631x over previous
import functools

import jax
import jax.numpy as jnp
from jax import lax
from jax.experimental import pallas as pl
from jax.experimental.pallas import tpu as pltpu
from jax.experimental.pallas import tpu_sc as plsc

_G = 16


def _make_gather(B, D, num_cores, num_subcores):
    nw = num_cores * num_subcores
    b_per_w = B // nw
    n_groups = b_per_w // _G
    mesh = plsc.VectorSubcoreMesh(core_axis_name="c", subcore_axis_name="s")

    @functools.partial(
        pl.kernel,
        mesh=mesh,
        compiler_params=pltpu.CompilerParams(needs_layout_passes=False),
        out_type=jax.ShapeDtypeStruct((B, D), jnp.float32),
        scratch_types=[
            pltpu.VMEM((b_per_w,), jnp.int32),
            pltpu.VMEM((b_per_w, D), jnp.float32),
            pltpu.SemaphoreType.DMA,
            pltpu.SemaphoreType.DMA,
        ],
    )
    def gather(idx_hbm, table_hbm, out_hbm, idx_v, rows_v, isem, sem):
        wid = lax.axis_index("s") * num_cores + lax.axis_index("c")
        base = wid * b_per_w
        pltpu.async_copy(idx_hbm.at[pl.ds(base, b_per_w)], idx_v, isem).wait()

        def fire(g):
            v = idx_v[pl.ds(g * _G, _G)]
            return [
                pltpu.async_copy(
                    table_hbm.at[v[i]],
                    rows_v.at[g * _G + i],
                    sem,
                )
                for i in range(_G)
            ]

        pending = fire(0)
        for g in range(n_groups):
            nxt = fire(g + 1) if g + 1 < n_groups else []
            for c in pending:
                c.wait()
            pending = nxt

        pltpu.async_copy(rows_v, out_hbm.at[pl.ds(base, b_per_w)], isem).wait()

    return gather


def _proj_body(x_ref, wt_ref, b_ref, out_ref):
    out_ref[...] = (
        jnp.dot(x_ref[...], wt_ref[...], preferred_element_type=jnp.float32)
        + b_ref[...]
    )


def _proj(embs, Wt, b2d):
    B, D = embs.shape
    O = Wt.shape[1]
    blk = 2048
    return pl.pallas_call(
        _proj_body,
        grid=(B // blk,),
        in_specs=[
            pl.BlockSpec((blk, D), lambda i: (i, 0)),
            pl.BlockSpec((D, O), lambda i: (0, 0)),
            pl.BlockSpec((1, O), lambda i: (0, 0)),
        ],
        out_specs=pl.BlockSpec((blk, O), lambda i: (i, 0)),
        out_shape=jax.ShapeDtypeStruct((B, O), jnp.float32),
    )(embs, Wt, b2d)


def kernel(indices, table, W, b):
    info = plsc.get_sparse_core_info()
    V, D = table.shape
    embs = _make_gather(indices.shape[0], D,
                        info.num_cores, info.num_subcores)(
        indices.astype(jnp.int32), table
    )
    return _proj(embs, W.T, b.reshape(1, -1))


# speedup vs baseline: 1.7540x; 1.0045x over previous
import functools

import jax
import jax.numpy as jnp
from jax import lax
from jax.experimental import pallas as pl
from jax.experimental.pallas import tpu as pltpu
from jax.experimental.pallas import tpu_sc as plsc

_G = 16


def _make_gather(B, D, num_cores, num_subcores):
    nw = num_cores * num_subcores
    b_per_w = B // nw
    n_groups = b_per_w // _G
    mesh = plsc.VectorSubcoreMesh(core_axis_name="c", subcore_axis_name="s")

    @functools.partial(
        pl.kernel,
        mesh=mesh,
        compiler_params=pltpu.CompilerParams(
            needs_layout_passes=False, use_tc_tiling_on_sc=True
        ),
        out_type=jax.ShapeDtypeStruct((B, D), jnp.float32),
        scratch_types=[
            pltpu.VMEM((b_per_w,), jnp.int32),
            pltpu.VMEM((b_per_w, D), jnp.float32),
            pltpu.SemaphoreType.DMA,
            pltpu.SemaphoreType.DMA,
        ],
    )
    def gather(idx_hbm, table_hbm, out_hbm, idx_v, rows_v, isem, sem):
        wid = lax.axis_index("s") * num_cores + lax.axis_index("c")
        base = wid * b_per_w
        pltpu.async_copy(idx_hbm.at[pl.ds(base, b_per_w)], idx_v, isem).wait()

        def fire(g):
            v = idx_v[pl.ds(g * _G, _G)]
            return [
                pltpu.async_copy(
                    table_hbm.at[v[i]],
                    rows_v.at[g * _G + i],
                    sem,
                )
                for i in range(_G)
            ]

        pending = fire(0)
        for g in range(n_groups):
            nxt = fire(g + 1) if g + 1 < n_groups else []
            for c in pending:
                c.wait()
            pending = nxt

        pltpu.async_copy(rows_v, out_hbm.at[pl.ds(base, b_per_w)], isem).wait()

    return gather


def _proj_body(x_ref, wt_ref, b_ref, out_ref):
    out_ref[...] = (
        jnp.dot(x_ref[...], wt_ref[...], preferred_element_type=jnp.float32)
        + b_ref[...]
    )


def _proj(embs, Wt, b2d):
    B, D = embs.shape
    O = Wt.shape[1]
    blk = 2048
    return pl.pallas_call(
        _proj_body,
        grid=(B // blk,),
        in_specs=[
            pl.BlockSpec((blk, D), lambda i: (i, 0)),
            pl.BlockSpec((D, O), lambda i: (0, 0)),
            pl.BlockSpec((1, O), lambda i: (0, 0)),
        ],
        out_specs=pl.BlockSpec((blk, O), lambda i: (i, 0)),
        out_shape=jax.ShapeDtypeStruct((B, O), jnp.float32),
    )(embs, Wt, b2d)


def kernel(indices, table, W, b):
    info = plsc.get_sparse_core_info()
    V, D = table.shape
    embs = _make_gather(indices.shape[0], D,
                        info.num_cores, info.num_subcores)(
        indices.astype(jnp.int32), table
    )
    return _proj(embs, W.T, b.reshape(1, -1))
